# Initial kernel scaffold; baseline (speedup 1.0000x reference)
import jax
import jax.numpy as jnp
from jax.experimental import pallas as pl


def kernel(x, W1, b1, W2, b2, W3, b3, W4, b4, W5, b5):
    raise NotImplementedError("write your pallas kernel here")


# speedup vs baseline: 91.9626x; 91.9626x over previous
import jax
import jax.numpy as jnp
from jax.experimental import pallas as pl

_K = 16
_P = 256
_B = 8


def _gcn_tower_kernel(x_ref, W1_ref, b1_ref, W2_ref, b2_ref, W3_ref, b3_ref,
                      out_ref):
    hs = [x_ref[bb] for bb in range(_B)]
    cols = jax.lax.broadcasted_iota(jnp.int32, (_B * _P, _P), 1)
    rows = jax.lax.broadcasted_iota(jnp.int32, (_P, _P), 0)
    eye = rows == jax.lax.broadcasted_iota(jnp.int32, (_P, _P), 1)
    eye_f = eye.astype(jnp.float32)
    ones_col = jnp.ones((_P, 1), jnp.float32)
    cn = (((1,), (1,)), ((), ()))
    ct = (((0,), (0,)), ((), ()))

    for (W_ref, b_ref) in ((W1_ref, b1_ref), (W2_ref, b2_ref), (W3_ref, b3_ref)):
        W = W_ref[...]
        bias = b_ref[...]

        cs = []
        for bb in range(_B):
            h = hs[bb]
            G = jax.lax.dot_general(h, h, cn, preferred_element_type=jnp.float32)
            sq_col = jnp.sum(h * h, axis=1, keepdims=True)
            sq_row = jax.lax.dot_general(sq_col, eye_f, ct,
                                         preferred_element_type=jnp.float32)
            d2 = sq_col + sq_row - 2.0 * G
            cs.append(jnp.sqrt(jnp.maximum(d2, 0.0)))
        c_all = jnp.concatenate(cs, axis=0)

        def step(t, carry):
            M, rem = carry
            rowmin = jnp.min(rem, axis=1, keepdims=True)
            ismin = rem <= rowmin
            minidx = jnp.min(jnp.where(ismin, cols, _P), axis=1, keepdims=True)
            sel = cols == minidx
            keep = jnp.where(t > 0, 1.0, 0.0)
            M = M + jnp.where(sel, keep, 0.0)
            rem = jnp.where(sel, jnp.inf, rem)
            return M, rem

        M_all, _ = jax.lax.fori_loop(
            0, _K + 1, step,
            (jnp.zeros((_B * _P, _P), jnp.float32), c_all))

        Msum = M_all[0 * _P:1 * _P]
        for bb in range(1, _B):
            Msum = Msum + M_all[bb * _P:(bb + 1) * _P]

        deg = 1.0 + jax.lax.dot_general(Msum, ones_col, ct,
                                        preferred_element_type=jnp.float32)
        dinv = jax.lax.rsqrt(deg)

        h_stack = jnp.concatenate(hs, axis=0)
        xl = jax.lax.dot_general(h_stack, W, cn,
                                 preferred_element_type=jnp.float32)
        new_hs = []
        for bb in range(_B):
            y = xl[bb * _P:(bb + 1) * _P] * dinv
            z = jax.lax.dot_general(Msum, y, ct,
                                    preferred_element_type=jnp.float32)
            new_hs.append(jnp.maximum((z + y) * dinv + bias, 0.0))
        hs = new_hs

    for bb in range(_B):
        out_ref[bb] = hs[bb]


def _mlp_kernel(h_ref, W4_ref, b4_ref, W5_ref, b5_ref, out_ref):
    cn = (((1,), (1,)), ((), ()))
    t = jax.lax.dot_general(h_ref[...], W4_ref[...], cn,
                            preferred_element_type=jnp.float32)
    t = jnp.maximum(t + b4_ref[...], 0.0)
    o = jax.lax.dot_general(t, W5_ref[...], cn,
                            preferred_element_type=jnp.float32) + b5_ref[...]
    o = o - jnp.max(o, axis=1, keepdims=True)
    e = jnp.exp(o)
    out_ref[...] = e / jnp.sum(e, axis=1, keepdims=True)


def kernel(x, W1, b1, W2, b2, W3, b3, W4, b4, W5, b5):
    B = x.shape[0]
    full = lambda s: pl.BlockSpec(s, lambda *a: (0,) * len(s))
    h3 = pl.pallas_call(
        _gcn_tower_kernel,
        in_specs=[
            full(x.shape),
            full(W1.shape), full((1, b1.shape[0])),
            full(W2.shape), full((1, b2.shape[0])),
            full(W3.shape), full((1, b3.shape[0])),
        ],
        out_specs=full((B, _P, W3.shape[0])),
        out_shape=jax.ShapeDtypeStruct((B, _P, W3.shape[0]), jnp.float32),
    )(x, W1, b1.reshape(1, -1), W2, b2.reshape(1, -1), W3, b3.reshape(1, -1))

    hflat = h3.reshape(B, -1)
    out = pl.pallas_call(
        _mlp_kernel,
        in_specs=[
            full(hflat.shape), full(W4.shape), full((1, b4.shape[0])),
            full(W5.shape), full((1, b5.shape[0])),
        ],
        out_specs=full((B, W5.shape[0])),
        out_shape=jax.ShapeDtypeStruct((B, W5.shape[0]), jnp.float32),
    )(hflat, W4, b4.reshape(1, -1), W5, b5.reshape(1, -1))
    return out
